# baseline (speedup 1.0000x reference)
import functools

import jax
import jax.numpy as jnp
from jax import lax
from jax.experimental import pallas as pl
from jax.experimental.pallas import tpu as pltpu
from jax.experimental.pallas import tpu_sc as plsc

B, C, N = 16, 64, 2048
OUT, EMB = 64, 128
K = 20
KPAD = 32
TN = 128
NB = N // TN
EPS = 1e-5
BIGF = 3.0e38

NC, NS = 2, 16
NW = NC * NS
QB = 64
NQB = (B * N) // QB
QB_PER_W = NQB // NW
QB_PER_BATCH = N // QB


def _knn_body(xb_ref, xf_ref, cw_ref, idx_ref, y1_ref, y2_ref, d2_ref, sel_ref):
    b = pl.program_id(0)
    nblk = pl.program_id(1)
    xb = xb_ref[0]
    xf = xf_ref[0]

    w1 = cw_ref[:, :C]
    w2m = cw_ref[:, C:] - w1
    y1_ref[0] = lax.dot_general(xb, w1, (((0,), (1,)), ((), ())),
                                preferred_element_type=jnp.float32, precision=lax.Precision.HIGHEST)
    y2_ref[0] = lax.dot_general(xb, w2m, (((0,), (1,)), ((), ())),
                                preferred_element_type=jnp.float32, precision=lax.Precision.HIGHEST)

    g = lax.dot_general(xb.astype(jnp.bfloat16), xf.astype(jnp.bfloat16),
                        (((0,), (0,)), ((), ())),
                        preferred_element_type=jnp.float32)
    sqf = jnp.sum(xf * xf, axis=0, keepdims=True)
    d2 = sqf - 2.0 * g

    col_i = lax.broadcasted_iota(jnp.int32, (TN, N), 1)
    col_f = col_i.astype(jnp.float32)
    row_g = lax.broadcasted_iota(jnp.int32, (TN, N), 0) + nblk * TN
    d2_ref[...] = jnp.where(col_i == row_g, BIGF, d2)

    lane_k = lax.broadcasted_iota(jnp.int32, (TN, KPAD), 1)

    def step(t, _):
        d2v = d2_ref[...]
        rmin = jnp.min(d2v, axis=1, keepdims=True)
        key = jnp.where(d2v <= rmin, col_f, float(N))
        amin = jnp.min(key, axis=1, keepdims=True)
        d2_ref[...] = jnp.where(col_f == amin, BIGF, d2v)
        sel_ref[...] = jnp.where(lane_k == t, amin, sel_ref[...])
        return 0

    lax.fori_loop(0, K, step, 0)

    ident = (lax.broadcasted_iota(jnp.int32, (TN, TN), 0)
             == lax.broadcasted_iota(jnp.int32, (TN, TN), 1)).astype(jnp.float32)
    sel_t = lax.dot_general(sel_ref[...], ident, (((0,), (0,)), ((), ())),
                            preferred_element_type=jnp.float32, precision=lax.Precision.HIGHEST)
    idx_ref[0] = (sel_t[:K] + jnp.float32(b * N)).astype(jnp.int32)


def _build_graph(x, conv_w):
    bh = x.shape[0]
    grid = (bh, NB)
    return pl.pallas_call(
        _knn_body,
        grid=grid,
        in_specs=[
            pl.BlockSpec((1, C, TN), lambda b, n: (b, 0, n)),
            pl.BlockSpec((1, C, N), lambda b, n: (b, 0, 0)),
            pl.BlockSpec((OUT, 2 * C), lambda b, n: (0, 0)),
        ],
        out_specs=[
            pl.BlockSpec((1, K, TN), lambda b, n: (b, 0, n)),
            pl.BlockSpec((1, TN, OUT), lambda b, n: (b, n, 0)),
            pl.BlockSpec((1, TN, OUT), lambda b, n: (b, n, 0)),
        ],
        out_shape=[
            jax.ShapeDtypeStruct((bh, K, N), jnp.int32),
            jax.ShapeDtypeStruct((bh, N, OUT), jnp.float32),
            jax.ShapeDtypeStruct((bh, N, OUT), jnp.float32),
        ],
        scratch_shapes=[
            pltpu.VMEM((TN, N), jnp.float32),
            pltpu.VMEM((TN, KPAD), jnp.float32),
        ],
    )(x, x, conv_w)


def _sc_body(qb_per_w, y1_hbm, idx_hbm, out_hbm, idx_v, rows_v, out_v, sem):
    wid = lax.axis_index("s") * NC + lax.axis_index("c")

    def block(i, _):
        qb = wid * qb_per_w + i
        b = lax.shift_right_logical(qb, 5)
        q0 = lax.shift_left(qb & (QB_PER_BATCH - 1), 6)
        q0 = pl.multiple_of(q0, QB)
        pltpu.sync_copy(idx_hbm.at[pl.ds(b * K, K), pl.ds(q0, QB)], idx_v)
        cps = []
        for t in range(K):
            cps.append(pltpu.async_copy(y1_hbm.at[idx_v.at[t]], rows_v.at[t], sem))
        for cp in cps:
            cp.wait()

        def chunk(r, _):
            for j in range(OUT // 16):
                acc = rows_v[0, r, pl.ds(j * 16, 16)]
                for t in range(1, K):
                    acc = jnp.maximum(acc, rows_v[t, r, pl.ds(j * 16, 16)])
                out_v[r, pl.ds(j * 16, 16)] = acc
            return 0

        lax.fori_loop(0, QB, chunk, 0)
        pltpu.sync_copy(out_v, out_hbm.at[pl.ds(qb * QB, QB), :])
        return 0

    lax.fori_loop(0, qb_per_w, block, 0)


def _sc_gather_max(y1_flat, idx_flat):
    rows = y1_flat.shape[0]
    qb_per_w = (rows // QB) // NW
    mesh = plsc.VectorSubcoreMesh(core_axis_name="c", subcore_axis_name="s")
    f = functools.partial(
        pl.kernel,
        out_type=jax.ShapeDtypeStruct((rows, OUT), jnp.float32),
        mesh=mesh,
        compiler_params=pltpu.CompilerParams(use_tc_tiling_on_sc=False),
        scratch_types=[
            pltpu.VMEM((K, QB), jnp.int32),
            pltpu.VMEM((K, QB, OUT), jnp.float32),
            pltpu.VMEM((QB, OUT), jnp.float32),
            pltpu.SemaphoreType.DMA,
        ],
    )(functools.partial(_sc_body, qb_per_w))
    return f(y1_flat, idx_flat)


def _norm_body(mx_ref, y2_ref, emb_ref, gw1_ref, gb1_ref, gw2_ref, gb2_ref,
               bw1_ref, bb1_ref, bw2_ref, bb2_ref, out_ref):
    pre = mx_ref[0] + y2_ref[0]
    mean = jnp.mean(pre, axis=0, keepdims=True)
    cen = pre - mean
    var = jnp.mean(cen * cen, axis=0, keepdims=True)
    inv = 1.0 / jnp.sqrt(var + EPS)

    e = emb_ref[0]
    hg = jnp.maximum(
        lax.dot_general(e, gw1_ref[...], (((1,), (1,)), ((), ())),
                        preferred_element_type=jnp.float32, precision=lax.Precision.HIGHEST) + gb1_ref[...], 0.0)
    gam = 1.0 + lax.dot_general(hg, gw2_ref[...], (((1,), (1,)), ((), ())),
                                preferred_element_type=jnp.float32, precision=lax.Precision.HIGHEST) + gb2_ref[...]
    hb = jnp.maximum(
        lax.dot_general(e, bw1_ref[...], (((1,), (1,)), ((), ())),
                        preferred_element_type=jnp.float32, precision=lax.Precision.HIGHEST) + bb1_ref[...], 0.0)
    bet = lax.dot_general(hb, bw2_ref[...], (((1,), (1,)), ((), ())),
                          preferred_element_type=jnp.float32, precision=lax.Precision.HIGHEST) + bb2_ref[...]

    res = jnp.maximum(cen * inv * gam + bet, 0.0)
    ident = (lax.broadcasted_iota(jnp.int32, (OUT, OUT), 0)
             == lax.broadcasted_iota(jnp.int32, (OUT, OUT), 1)).astype(jnp.float32)
    out_ref[0] = lax.dot_general(ident, res, (((1,), (1,)), ((), ())),
                                 preferred_element_type=jnp.float32, precision=lax.Precision.HIGHEST)


def _norm_film(mx, y2, emb3, g_w1, g_b1, g_w2, g_b2, b_w1, b_b1, b_w2, b_b2):
    full = lambda *s: pl.BlockSpec(s, lambda b: tuple(0 for _ in s))
    return pl.pallas_call(
        _norm_body,
        grid=(mx.shape[0],),
        in_specs=[
            pl.BlockSpec((1, N, OUT), lambda b: (b, 0, 0)),
            pl.BlockSpec((1, N, OUT), lambda b: (b, 0, 0)),
            pl.BlockSpec((1, 1, EMB), lambda b: (b, 0, 0)),
            full(EMB, EMB), full(1, EMB), full(OUT, EMB), full(1, OUT),
            full(EMB, EMB), full(1, EMB), full(OUT, EMB), full(1, OUT),
        ],
        out_specs=pl.BlockSpec((1, OUT, N), lambda b: (b, 0, 0)),
        out_shape=jax.ShapeDtypeStruct((mx.shape[0], OUT, N), jnp.float32),
    )(mx, y2, emb3, g_w1, g_b1, g_w2, g_b2, b_w1, b_b1, b_w2, b_b2)


def kernel(x, domain_emb, conv_w, g_w1, g_b1, g_w2, g_b2, b_w1, b_b1, b_w2, b_b2):
    h = B // 4
    outs = []
    for lo in range(0, B, h):
        idx, y1, y2 = _build_graph(x[lo:lo + h], conv_w)
        mx = _sc_gather_max(y1.reshape(h * N, OUT), idx.reshape(h * K, N))
        outs.append(_norm_film(
            mx.reshape(h, N, OUT), y2,
            domain_emb[lo:lo + h].reshape(h, 1, EMB),
            g_w1, g_b1.reshape(1, EMB), g_w2, g_b2.reshape(1, OUT),
            b_w1, b_b1.reshape(1, EMB), b_w2, b_b2.reshape(1, OUT)))
    return jnp.concatenate(outs, axis=0)


# speedup vs baseline: 1.1870x; 1.1870x over previous
import functools

import jax
import jax.numpy as jnp
from jax import lax
from jax.experimental import pallas as pl
from jax.experimental.pallas import tpu as pltpu
from jax.experimental.pallas import tpu_sc as plsc

B, C, N = 16, 64, 2048
OUT, EMB = 64, 128
K = 20
KPAD = 32
TN = 256
NB = N // TN
EPS = 1e-5
BIGF = 3.0e38

NC, NS = 2, 16
NW = NC * NS
QB = 64
NQB = (B * N) // QB
QB_PER_W = NQB // NW
QB_PER_BATCH = N // QB


def _knn_body(xb_ref, xf_ref, cw_ref, idx_ref, y1_ref, y2_ref, d2_ref, sel_ref):
    b = pl.program_id(0)
    nblk = pl.program_id(1)
    xb = xb_ref[0]
    xf = xf_ref[0]

    w1 = cw_ref[:, :C]
    w2m = cw_ref[:, C:] - w1
    y1_ref[0] = lax.dot_general(xb, w1, (((0,), (1,)), ((), ())),
                                preferred_element_type=jnp.float32, precision=lax.Precision.HIGHEST)
    y2_ref[0] = lax.dot_general(xb, w2m, (((0,), (1,)), ((), ())),
                                preferred_element_type=jnp.float32, precision=lax.Precision.HIGHEST)

    g = lax.dot_general(xb.astype(jnp.bfloat16), xf.astype(jnp.bfloat16),
                        (((0,), (0,)), ((), ())),
                        preferred_element_type=jnp.float32)
    sqf = jnp.sum(xf * xf, axis=0, keepdims=True)
    d2 = sqf - 2.0 * g

    col_i = lax.broadcasted_iota(jnp.int32, (TN, N), 1)
    col_f = col_i.astype(jnp.float32)
    row_g = lax.broadcasted_iota(jnp.int32, (TN, N), 0) + nblk * TN
    d2_ref[...] = jnp.where(col_i == row_g, BIGF, d2)

    lane_k = lax.broadcasted_iota(jnp.int32, (TN, KPAD), 1)

    def step(t, _):
        d2v = d2_ref[...]
        rmin = jnp.min(d2v, axis=1, keepdims=True)
        key = jnp.where(d2v <= rmin, col_f, float(N))
        amin = jnp.min(key, axis=1, keepdims=True)
        d2_ref[...] = jnp.where(col_f == amin, BIGF, d2v)
        sel_ref[...] = jnp.where(lane_k == t, amin, sel_ref[...])
        return 0

    lax.fori_loop(0, K, step, 0)

    ident = (lax.broadcasted_iota(jnp.int32, (TN, TN), 0)
             == lax.broadcasted_iota(jnp.int32, (TN, TN), 1)).astype(jnp.float32)
    sel_t = lax.dot_general(sel_ref[...], ident, (((0,), (0,)), ((), ())),
                            preferred_element_type=jnp.float32, precision=lax.Precision.HIGHEST)
    idx_ref[0] = (sel_t[:K] + jnp.float32(b * N)).astype(jnp.int32)


def _build_graph(x, conv_w):
    bh = x.shape[0]
    grid = (bh, NB)
    return pl.pallas_call(
        _knn_body,
        grid=grid,
        in_specs=[
            pl.BlockSpec((1, C, TN), lambda b, n: (b, 0, n)),
            pl.BlockSpec((1, C, N), lambda b, n: (b, 0, 0)),
            pl.BlockSpec((OUT, 2 * C), lambda b, n: (0, 0)),
        ],
        out_specs=[
            pl.BlockSpec((1, K, TN), lambda b, n: (b, 0, n)),
            pl.BlockSpec((1, TN, OUT), lambda b, n: (b, n, 0)),
            pl.BlockSpec((1, TN, OUT), lambda b, n: (b, n, 0)),
        ],
        out_shape=[
            jax.ShapeDtypeStruct((bh, K, N), jnp.int32),
            jax.ShapeDtypeStruct((bh, N, OUT), jnp.float32),
            jax.ShapeDtypeStruct((bh, N, OUT), jnp.float32),
        ],
        scratch_shapes=[
            pltpu.VMEM((TN, N), jnp.float32),
            pltpu.VMEM((TN, KPAD), jnp.float32),
        ],
    )(x, x, conv_w)


def _sc_body(qb_per_w, y1_hbm, idx_hbm, out_hbm, idx_v, rows_v, out_v, sem):
    wid = lax.axis_index("s") * NC + lax.axis_index("c")

    def block(i, _):
        qb = wid * qb_per_w + i
        b = lax.shift_right_logical(qb, 5)
        q0 = lax.shift_left(qb & (QB_PER_BATCH - 1), 6)
        q0 = pl.multiple_of(q0, QB)
        pltpu.sync_copy(idx_hbm.at[pl.ds(b * K, K), pl.ds(q0, QB)], idx_v)
        cps = []
        for t in range(K):
            cps.append(pltpu.async_copy(y1_hbm.at[idx_v.at[t]], rows_v.at[t], sem))
        for cp in cps:
            cp.wait()

        def chunk(r, _):
            for j in range(OUT // 16):
                acc = rows_v[0, r, pl.ds(j * 16, 16)]
                for t in range(1, K):
                    acc = jnp.maximum(acc, rows_v[t, r, pl.ds(j * 16, 16)])
                out_v[r, pl.ds(j * 16, 16)] = acc
            return 0

        lax.fori_loop(0, QB, chunk, 0)
        pltpu.sync_copy(out_v, out_hbm.at[pl.ds(qb * QB, QB), :])
        return 0

    lax.fori_loop(0, qb_per_w, block, 0)


def _sc_gather_max(y1_flat, idx_flat):
    rows = y1_flat.shape[0]
    qb_per_w = (rows // QB) // NW
    mesh = plsc.VectorSubcoreMesh(core_axis_name="c", subcore_axis_name="s")
    f = functools.partial(
        pl.kernel,
        out_type=jax.ShapeDtypeStruct((rows, OUT), jnp.float32),
        mesh=mesh,
        compiler_params=pltpu.CompilerParams(use_tc_tiling_on_sc=False),
        scratch_types=[
            pltpu.VMEM((K, QB), jnp.int32),
            pltpu.VMEM((K, QB, OUT), jnp.float32),
            pltpu.VMEM((QB, OUT), jnp.float32),
            pltpu.SemaphoreType.DMA,
        ],
    )(functools.partial(_sc_body, qb_per_w))
    return f(y1_flat, idx_flat)


def _norm_body(mx_ref, y2_ref, emb_ref, gw1_ref, gb1_ref, gw2_ref, gb2_ref,
               bw1_ref, bb1_ref, bw2_ref, bb2_ref, out_ref):
    pre = mx_ref[0] + y2_ref[0]
    mean = jnp.mean(pre, axis=0, keepdims=True)
    cen = pre - mean
    var = jnp.mean(cen * cen, axis=0, keepdims=True)
    inv = 1.0 / jnp.sqrt(var + EPS)

    e = emb_ref[0]
    hg = jnp.maximum(
        lax.dot_general(e, gw1_ref[...], (((1,), (1,)), ((), ())),
                        preferred_element_type=jnp.float32, precision=lax.Precision.HIGHEST) + gb1_ref[...], 0.0)
    gam = 1.0 + lax.dot_general(hg, gw2_ref[...], (((1,), (1,)), ((), ())),
                                preferred_element_type=jnp.float32, precision=lax.Precision.HIGHEST) + gb2_ref[...]
    hb = jnp.maximum(
        lax.dot_general(e, bw1_ref[...], (((1,), (1,)), ((), ())),
                        preferred_element_type=jnp.float32, precision=lax.Precision.HIGHEST) + bb1_ref[...], 0.0)
    bet = lax.dot_general(hb, bw2_ref[...], (((1,), (1,)), ((), ())),
                          preferred_element_type=jnp.float32, precision=lax.Precision.HIGHEST) + bb2_ref[...]

    res = jnp.maximum(cen * inv * gam + bet, 0.0)
    ident = (lax.broadcasted_iota(jnp.int32, (OUT, OUT), 0)
             == lax.broadcasted_iota(jnp.int32, (OUT, OUT), 1)).astype(jnp.float32)
    out_ref[0] = lax.dot_general(ident, res, (((1,), (1,)), ((), ())),
                                 preferred_element_type=jnp.float32, precision=lax.Precision.HIGHEST)


def _norm_film(mx, y2, emb3, g_w1, g_b1, g_w2, g_b2, b_w1, b_b1, b_w2, b_b2):
    full = lambda *s: pl.BlockSpec(s, lambda b: tuple(0 for _ in s))
    return pl.pallas_call(
        _norm_body,
        grid=(mx.shape[0],),
        in_specs=[
            pl.BlockSpec((1, N, OUT), lambda b: (b, 0, 0)),
            pl.BlockSpec((1, N, OUT), lambda b: (b, 0, 0)),
            pl.BlockSpec((1, 1, EMB), lambda b: (b, 0, 0)),
            full(EMB, EMB), full(1, EMB), full(OUT, EMB), full(1, OUT),
            full(EMB, EMB), full(1, EMB), full(OUT, EMB), full(1, OUT),
        ],
        out_specs=pl.BlockSpec((1, OUT, N), lambda b: (b, 0, 0)),
        out_shape=jax.ShapeDtypeStruct((mx.shape[0], OUT, N), jnp.float32),
    )(mx, y2, emb3, g_w1, g_b1, g_w2, g_b2, b_w1, b_b1, b_w2, b_b2)


def kernel(x, domain_emb, conv_w, g_w1, g_b1, g_w2, g_b2, b_w1, b_b1, b_w2, b_b2):
    h = B // 4
    outs = []
    for lo in range(0, B, h):
        idx, y1, y2 = _build_graph(x[lo:lo + h], conv_w)
        mx = _sc_gather_max(y1.reshape(h * N, OUT), idx.reshape(h * K, N))
        outs.append(_norm_film(
            mx.reshape(h, N, OUT), y2,
            domain_emb[lo:lo + h].reshape(h, 1, EMB),
            g_w1, g_b1.reshape(1, EMB), g_w2, g_b2.reshape(1, OUT),
            b_w1, b_b1.reshape(1, EMB), b_w2, b_b2.reshape(1, OUT)))
    return jnp.concatenate(outs, axis=0)
